# Initial kernel scaffold; baseline (speedup 1.0000x reference)
import jax
import jax.numpy as jnp
from jax.experimental import pallas as pl


def kernel(x, edge_index, W1, b1, W2, b2):
    raise NotImplementedError("write your pallas kernel here")


# speedup vs baseline: 19.3770x; 19.3770x over previous
import functools
import jax
import jax.numpy as jnp
from jax import lax
from jax.experimental import pallas as pl
from jax.experimental.pallas import tpu as pltpu
from jax.experimental.pallas import tpu_sc as plsc

N = 100000
E = 3200000
NPAD = 100096
EPAD = 3203072
ROWS = EPAD // 128
NC = 2
NS = 16
CH = 2
RPT = NPAD // NS

_MESH = plsc.VectorSubcoreMesh(core_axis_name="c", subcore_axis_name="s")
_SC_PARAMS = pltpu.CompilerParams(use_tc_tiling_on_sc=False)
_f32 = jnp.float32


@functools.partial(
    pl.kernel,
    out_type=jax.ShapeDtypeStruct((NC, NPAD, 8), _f32),
    mesh=_MESH,
    compiler_params=_SC_PARAMS,
    scratch_types=[
        pltpu.VMEM((CH, 128), jnp.int32),
        pltpu.VMEM((128, 8), _f32),
        pltpu.VMEM_SHARED((NPAD, 8), _f32),
    ],
)
def _deg_call(dst_hbm, ones_hbm, zeros8_hbm, out_hbm, dsti, onesv, acc):
    c = lax.axis_index("c")
    s = lax.axis_index("s")
    pltpu.sync_copy(zeros8_hbm.at[pl.ds(s * RPT, RPT)], acc.at[pl.ds(s * RPT, RPT)])
    pltpu.sync_copy(ones_hbm, onesv)
    plsc.subcore_barrier()
    tile_rows = ROWS // (NC * NS)
    base = (c * NS + s) * tile_rows

    def step(i, carry):
        pltpu.sync_copy(dst_hbm.at[pl.ds(base + i * CH, CH)], dsti)
        for j in range(CH):
            pltpu.sync_copy(onesv, acc.at[dsti.at[j]], add=True)
        return carry

    lax.fori_loop(0, tile_rows // CH, step, 0)
    plsc.subcore_barrier()
    pltpu.sync_copy(acc.at[pl.ds(s * RPT, RPT)],
                    out_hbm.at[c].at[pl.ds(s * RPT, RPT)])


def _make_agg(edge_split):
    ntab = 1 if edge_split else NC

    @functools.partial(
        pl.kernel,
        out_type=jax.ShapeDtypeStruct((NC, NPAD, 16), _f32),
        mesh=_MESH,
        compiler_params=_SC_PARAMS,
        scratch_types=[
            pltpu.VMEM((CH, 128), jnp.int32),
            pltpu.VMEM((CH, 128), jnp.int32),
            pltpu.VMEM((CH, 128, 16), _f32),
            pltpu.VMEM_SHARED((NPAD, 16), _f32),
            pltpu.SemaphoreType.DMA,
        ],
    )
    def agg(src_hbm, dst_hbm, gtab_hbm, zeros_hbm, out_hbm,
            srci, dsti, gbuf, acc, sem):
        c = lax.axis_index("c")
        s = lax.axis_index("s")
        pltpu.sync_copy(zeros_hbm.at[pl.ds(s * RPT, RPT)],
                        acc.at[pl.ds(s * RPT, RPT)])
        plsc.subcore_barrier()
        if edge_split:
            tile_rows = ROWS // (NC * NS)
            base = (c * NS + s) * tile_rows
            tab = gtab_hbm.at[0]
        else:
            tile_rows = ROWS // NS
            base = s * tile_rows
            tab = gtab_hbm.at[c]

        def step(i, carry):
            r = base + i * CH
            pltpu.sync_copy(src_hbm.at[pl.ds(r, CH)], srci)
            pltpu.sync_copy(dst_hbm.at[pl.ds(r, CH)], dsti)
            for j in range(CH):
                pltpu.async_copy(tab.at[srci.at[j]], gbuf.at[j], sem).wait()
                pltpu.sync_copy(gbuf.at[j], acc.at[dsti.at[j]], add=True)
            return carry

        lax.fori_loop(0, tile_rows // CH, step, 0)
        plsc.subcore_barrier()
        pltpu.sync_copy(acc.at[pl.ds(s * RPT, RPT)],
                        out_hbm.at[c].at[pl.ds(s * RPT, RPT)])

    return agg


_l1_call = _make_agg(edge_split=False)
_l2_call = _make_agg(edge_split=True)


def _tc1_body(x_ref, w1_ref, d0_ref, d1_ref, gab_ref, dis_ref):
    deg = d0_ref[...] + d1_ref[...] + 1.0
    dis = lax.rsqrt(jnp.maximum(deg, 1e-12))
    h = jnp.dot(x_ref[...], w1_ref[...], preferred_element_type=_f32)
    g = h * dis
    gab_ref[0] = g[:, :16]
    gab_ref[1] = g[:, 16:]
    dis_ref[...] = dis


def _tc2_body(s1_ref, gab_ref, dis_ref, b1a_ref, b1b_ref, w2a_ref, w2b_ref,
              g2_ref):
    dis = dis_ref[...]
    h1a = jnp.maximum(dis * (s1_ref[0] + gab_ref[0]) + b1a_ref[...], 0.0)
    h1b = jnp.maximum(dis * (s1_ref[1] + gab_ref[1]) + b1b_ref[...], 0.0)
    h2 = (jnp.dot(h1a, w2a_ref[...], preferred_element_type=_f32)
          + jnp.dot(h1b, w2b_ref[...], preferred_element_type=_f32))
    g2_ref[...] = h2 * dis


def _tc3_body(s2_ref, g2_ref, dis_ref, b2_ref, z_ref):
    z_ref[...] = (dis_ref[...] * (s2_ref[0] + s2_ref[1] + g2_ref[...])
                  + b2_ref[...])


_B = 3128
_GRID = NPAD // _B


def _rows3(i):
    return (0, i, 0)


def _rows2(i):
    return (i, 0)


def _full2(i):
    return (0, 0)


_blk3 = pl.BlockSpec((NC, _B, 16), _rows3)
_blk2 = pl.BlockSpec((_B, 16), _rows2)
_blk1 = pl.BlockSpec((_B, 1), _rows2)

_tc1 = pl.pallas_call(
    _tc1_body,
    grid=(_GRID,),
    in_specs=[pl.BlockSpec((_B, 6), _rows2),
              pl.BlockSpec((6, 32), _full2),
              _blk1, _blk1],
    out_specs=(_blk3, _blk1),
    out_shape=(jax.ShapeDtypeStruct((NC, NPAD, 16), _f32),
               jax.ShapeDtypeStruct((NPAD, 1), _f32)),
)
_tc2 = pl.pallas_call(
    _tc2_body,
    grid=(_GRID,),
    in_specs=[_blk3, _blk3, _blk1,
              pl.BlockSpec((1, 16), _full2), pl.BlockSpec((1, 16), _full2),
              pl.BlockSpec((16, 16), _full2), pl.BlockSpec((16, 16), _full2)],
    out_specs=_blk2,
    out_shape=jax.ShapeDtypeStruct((NPAD, 16), _f32),
)
_tc3 = pl.pallas_call(
    _tc3_body,
    grid=(_GRID,),
    in_specs=[_blk3, _blk2, _blk1, pl.BlockSpec((1, 16), _full2)],
    out_specs=_blk2,
    out_shape=jax.ShapeDtypeStruct((NPAD, 16), _f32),
)


@jax.jit
def kernel(x, edge_index, W1, b1, W2, b2):
    src = jnp.pad(edge_index[0], (0, EPAD - E), constant_values=N)
    dst = jnp.pad(edge_index[1], (0, EPAD - E), constant_values=N)
    src = src.reshape(ROWS, 128).astype(jnp.int32)
    dst = dst.reshape(ROWS, 128).astype(jnp.int32)
    xp = jnp.pad(x, ((0, NPAD - N), (0, 0)))
    zeros16 = jnp.zeros((NPAD, 16), _f32)
    zeros8 = jnp.zeros((NPAD, 8), _f32)
    ones = jnp.ones((128, 8), _f32)

    degp = _deg_call(dst, ones, zeros8)
    gab, dis = _tc1(xp, W1, degp[0, :, :1], degp[1, :, :1])
    s1 = _l1_call(src, dst, gab, zeros16)
    g2 = _tc2(s1, gab, dis, b1[:16][None, :], b1[16:][None, :],
              W2[:16], W2[16:])
    s2 = _l2_call(src, dst, g2[None], zeros16)
    z = _tc3(s2, g2, dis, b2[None, :])
    return z[:N]


# speedup vs baseline: 38.2969x; 1.9764x over previous
import functools
import jax
import jax.numpy as jnp
from jax import lax
from jax.experimental import pallas as pl
from jax.experimental.pallas import tpu as pltpu
from jax.experimental.pallas import tpu_sc as plsc

N = 100000
E = 3200000
NPAD = 100096
EPAD = 3211264
ROWS = EPAD // 128
NC = 2
NS = 16
CH = 2
NB = 4
RPT = NPAD // NS

_MESH = plsc.VectorSubcoreMesh(core_axis_name="c", subcore_axis_name="s")
_SC_PARAMS = pltpu.CompilerParams(use_tc_tiling_on_sc=False)
_f32 = jnp.float32


def _make_edge_kernel(edge_split, gather, width):
    ntiles = NC * NS if edge_split else NS
    tile_rows = ROWS // ntiles
    T = tile_rows // CH
    G = T // NB

    scratch = [
        pltpu.VMEM((NB, CH, 128), jnp.int32),
        pltpu.VMEM((NB, CH, 128), jnp.int32),
        pltpu.VMEM((NB, CH, 128, width), _f32),
        pltpu.VMEM_SHARED((NPAD, width), _f32),
        pltpu.SemaphoreType.DMA,
        pltpu.SemaphoreType.DMA,
        pltpu.SemaphoreType.DMA,
    ]

    @functools.partial(
        pl.kernel,
        out_type=jax.ShapeDtypeStruct((NC, NPAD, width), _f32),
        mesh=_MESH,
        compiler_params=_SC_PARAMS,
        scratch_types=scratch,
    )
    def edge_kernel(src_hbm, dst_hbm, gtab_hbm, zeros_hbm, out_hbm,
                    srci, dsti, gbuf, acc, isem, gsem, ssem):
        c = lax.axis_index("c")
        s = lax.axis_index("s")
        pltpu.sync_copy(zeros_hbm.at[pl.ds(s * RPT, RPT)],
                        acc.at[pl.ds(s * RPT, RPT)])
        if not gather:
            pltpu.sync_copy(gtab_hbm.at[0].at[pl.ds(0, 128)], gbuf.at[0, 0])
        plsc.subcore_barrier()
        if edge_split:
            base = (c * NS + s) * tile_rows
            tab = gtab_hbm.at[0]
        else:
            base = s * tile_rows
            tab = gtab_hbm.at[c]

        def idx_start(t_rows, slot):
            r = base + t_rows
            pltpu.make_async_copy(dst_hbm.at[pl.ds(r, CH)], dsti.at[slot],
                                  isem).start()
            if gather:
                pltpu.make_async_copy(src_hbm.at[pl.ds(r, CH)], srci.at[slot],
                                      isem).start()

        def idx_wait(slot):
            pltpu.make_async_copy(dst_hbm.at[pl.ds(base, CH)], dsti.at[slot],
                                  isem).wait()
            if gather:
                pltpu.make_async_copy(src_hbm.at[pl.ds(base, CH)],
                                      srci.at[slot], isem).wait()

        def scat_drain(slot):
            for j in range(CH):
                sbuf = gbuf.at[slot, j] if gather else gbuf.at[0, 0]
                pltpu.make_async_copy(sbuf, acc.at[dsti.at[slot, j]],
                                      ssem).wait()

        idx_start(0, 0)

        def step(g, carry):
            for b in range(NB):
                p = b
                q = (p + 1) % NB
                t = g * NB + b
                if b == NB - 1:
                    scat_drain(q)
                else:
                    @pl.when(g >= 1)
                    def _():
                        scat_drain(q)
                idx_wait(p)
                if b == NB - 1:
                    @pl.when(g < G - 1)
                    def _():
                        idx_start((t + 1) * CH, q)
                else:
                    idx_start((t + 1) * CH, q)
                if gather:
                    for j in range(CH):
                        pltpu.make_async_copy(tab.at[srci.at[p, j]],
                                              gbuf.at[p, j], gsem).start()
                    for j in range(CH):
                        pltpu.make_async_copy(tab.at[srci.at[p, j]],
                                              gbuf.at[p, j], gsem).wait()
                for j in range(CH):
                    sbuf = gbuf.at[p, j] if gather else gbuf.at[0, 0]
                    pltpu.make_async_copy(sbuf, acc.at[dsti.at[p, j]],
                                          ssem).start(add=True)
            return carry

        lax.fori_loop(0, G, step, 0)
        for k in range(NB - 1):
            scat_drain((T - (NB - 1) + k) % NB)
        plsc.subcore_barrier()
        pltpu.sync_copy(acc.at[pl.ds(s * RPT, RPT)],
                        out_hbm.at[c].at[pl.ds(s * RPT, RPT)])

    return edge_kernel


_deg_call = _make_edge_kernel(edge_split=True, gather=False, width=8)
_l1_call = _make_edge_kernel(edge_split=False, gather=True, width=16)
_l2_call = _make_edge_kernel(edge_split=True, gather=True, width=16)


def _tc1_body(x_ref, w1_ref, d0_ref, d1_ref, gab_ref, dis_ref):
    deg = d0_ref[...] + d1_ref[...] + 1.0
    dis = lax.rsqrt(jnp.maximum(deg, 1e-12))
    h = jnp.dot(x_ref[...], w1_ref[...], preferred_element_type=_f32)
    g = h * dis
    gab_ref[0] = g[:, :16]
    gab_ref[1] = g[:, 16:]
    dis_ref[...] = dis


def _tc2_body(s1_ref, gab_ref, dis_ref, b1a_ref, b1b_ref, w2a_ref, w2b_ref,
              g2_ref):
    dis = dis_ref[...]
    h1a = jnp.maximum(dis * (s1_ref[0] + gab_ref[0]) + b1a_ref[...], 0.0)
    h1b = jnp.maximum(dis * (s1_ref[1] + gab_ref[1]) + b1b_ref[...], 0.0)
    h2 = (jnp.dot(h1a, w2a_ref[...], preferred_element_type=_f32)
          + jnp.dot(h1b, w2b_ref[...], preferred_element_type=_f32))
    g2_ref[...] = h2 * dis


def _tc3_body(s2_ref, g2_ref, dis_ref, b2_ref, z_ref):
    z_ref[...] = (dis_ref[...] * (s2_ref[0] + s2_ref[1] + g2_ref[...])
                  + b2_ref[...])


_B = 3128
_GRID = NPAD // _B


def _rows3(i):
    return (0, i, 0)


def _rows2(i):
    return (i, 0)


def _full2(i):
    return (0, 0)


_blk3 = pl.BlockSpec((NC, _B, 16), _rows3)
_blk2 = pl.BlockSpec((_B, 16), _rows2)
_blk1 = pl.BlockSpec((_B, 1), _rows2)

_tc1 = pl.pallas_call(
    _tc1_body,
    grid=(_GRID,),
    in_specs=[pl.BlockSpec((_B, 6), _rows2),
              pl.BlockSpec((6, 32), _full2),
              _blk1, _blk1],
    out_specs=(_blk3, _blk1),
    out_shape=(jax.ShapeDtypeStruct((NC, NPAD, 16), _f32),
               jax.ShapeDtypeStruct((NPAD, 1), _f32)),
)
_tc2 = pl.pallas_call(
    _tc2_body,
    grid=(_GRID,),
    in_specs=[_blk3, _blk3, _blk1,
              pl.BlockSpec((1, 16), _full2), pl.BlockSpec((1, 16), _full2),
              pl.BlockSpec((16, 16), _full2), pl.BlockSpec((16, 16), _full2)],
    out_specs=_blk2,
    out_shape=jax.ShapeDtypeStruct((NPAD, 16), _f32),
)
_tc3 = pl.pallas_call(
    _tc3_body,
    grid=(_GRID,),
    in_specs=[_blk3, _blk2, _blk1, pl.BlockSpec((1, 16), _full2)],
    out_specs=_blk2,
    out_shape=jax.ShapeDtypeStruct((NPAD, 16), _f32),
)


@jax.jit
def kernel(x, edge_index, W1, b1, W2, b2):
    src = jnp.pad(edge_index[0], (0, EPAD - E), constant_values=N)
    dst = jnp.pad(edge_index[1], (0, EPAD - E), constant_values=N)
    src = src.reshape(ROWS, 128).astype(jnp.int32)
    dst = dst.reshape(ROWS, 128).astype(jnp.int32)
    xp = jnp.pad(x, ((0, NPAD - N), (0, 0)))
    zeros16 = jnp.zeros((NPAD, 16), _f32)
    zeros8 = jnp.zeros((NPAD, 8), _f32)
    ones = jnp.ones((1, 128, 8), _f32)

    degp = _deg_call(src, dst, ones, zeros8)
    gab, dis = _tc1(xp, W1, degp[0, :, :1], degp[1, :, :1])
    s1 = _l1_call(src, dst, gab, zeros16)
    g2 = _tc2(s1, gab, dis, b1[:16][None, :], b1[16:][None, :],
              W2[:16], W2[16:])
    s2 = _l2_call(src, dst, g2[None], zeros16)
    z = _tc3(s2, g2, dis, b2[None, :])
    return z[:N]


# speedup vs baseline: 42.2289x; 1.1027x over previous
import functools
import jax
import jax.numpy as jnp
from jax import lax
from jax.experimental import pallas as pl
from jax.experimental.pallas import tpu as pltpu
from jax.experimental.pallas import tpu_sc as plsc

N = 100000
E = 3200000
NPAD = 100096
EPAD = 3211264
ROWS = EPAD // 128
NC = 2
NS = 16
CH = 2
NB = 4
RPT = NPAD // NS

_MESH = plsc.VectorSubcoreMesh(core_axis_name="c", subcore_axis_name="s")
_SC_PARAMS = pltpu.CompilerParams(use_tc_tiling_on_sc=False)
_f32 = jnp.float32


def _make_edge_kernel(edge_split, gather, width):
    ntiles = NC * NS if edge_split else NS
    tile_rows = ROWS // ntiles
    T = tile_rows // CH
    G = T // NB

    scratch = [
        pltpu.VMEM((NB, CH, 128), jnp.int32),
        pltpu.VMEM((NB, CH, 128), jnp.int32),
        pltpu.VMEM((NB, CH, 128, width), _f32),
        pltpu.VMEM_SHARED((NPAD, width), _f32),
        pltpu.SemaphoreType.DMA,
        pltpu.SemaphoreType.DMA,
        pltpu.SemaphoreType.DMA,
    ]

    @functools.partial(
        pl.kernel,
        out_type=jax.ShapeDtypeStruct((NC, NPAD, width), _f32),
        mesh=_MESH,
        compiler_params=_SC_PARAMS,
        scratch_types=scratch,
    )
    def edge_kernel(src_hbm, dst_hbm, gtab_hbm, zeros_hbm, out_hbm,
                    srci, dsti, gbuf, acc, isem, gsem, ssem):
        c = lax.axis_index("c")
        s = lax.axis_index("s")
        pltpu.sync_copy(zeros_hbm.at[pl.ds(s * RPT, RPT)],
                        acc.at[pl.ds(s * RPT, RPT)])
        if not gather:
            pltpu.sync_copy(gtab_hbm.at[0].at[pl.ds(0, 128)], gbuf.at[0, 0])
        plsc.subcore_barrier()
        if edge_split:
            base = (c * NS + s) * tile_rows
            tab = gtab_hbm.at[0]
        else:
            base = s * tile_rows
            tab = gtab_hbm.at[c]

        def idx_start(t_rows, slot):
            r = base + t_rows
            pltpu.make_async_copy(dst_hbm.at[pl.ds(r, CH)], dsti.at[slot],
                                  isem).start()
            if gather:
                pltpu.make_async_copy(src_hbm.at[pl.ds(r, CH)], srci.at[slot],
                                      isem).start()

        def idx_wait(slot):
            pltpu.make_async_copy(dst_hbm.at[pl.ds(base, CH)], dsti.at[slot],
                                  isem).wait()
            if gather:
                pltpu.make_async_copy(src_hbm.at[pl.ds(base, CH)],
                                      srci.at[slot], isem).wait()

        def scat_drain(slot):
            for j in range(CH):
                sbuf = gbuf.at[slot, j] if gather else gbuf.at[0, 0]
                pltpu.make_async_copy(sbuf, acc.at[dsti.at[slot, j]],
                                      ssem).wait()

        idx_start(0, 0)
        idx_start(CH, 1)
        idx_wait(0)
        if gather:
            for j in range(CH):
                pltpu.make_async_copy(tab.at[srci.at[0, j]],
                                      gbuf.at[0, j], gsem).start()

        def step(g, carry):
            for b in range(NB):
                p = b
                q1 = (p + 1) % NB
                q2 = (p + 2) % NB
                t = g * NB + b

                def drain_a():
                    scat_drain(q2)

                if b >= 2:
                    drain_a()
                else:
                    @pl.when(g >= 1)
                    def _():
                        drain_a()

                def head_bcd():
                    idx_wait(q1)
                    if gather:
                        for j in range(CH):
                            pltpu.make_async_copy(tab.at[srci.at[q1, j]],
                                                  gbuf.at[q1, j], gsem).start()

                def issue_c():
                    idx_start((t + 2) * CH, q2)

                if b == 3:
                    @pl.when(g < G - 1)
                    def _():
                        head_bcd()
                        issue_c()
                else:
                    head_bcd()
                    if b >= 2:
                        @pl.when(g < G - 1)
                        def _():
                            issue_c()
                    else:
                        issue_c()

                if gather:
                    for j in range(CH):
                        pltpu.make_async_copy(tab.at[srci.at[p, j]],
                                              gbuf.at[p, j], gsem).wait()
                for j in range(CH):
                    sbuf = gbuf.at[p, j] if gather else gbuf.at[0, 0]
                    pltpu.make_async_copy(sbuf, acc.at[dsti.at[p, j]],
                                          ssem).start(add=True)
            return carry

        lax.fori_loop(0, G, step, 0)
        for k in range(2):
            scat_drain((T - 2 + k) % NB)
        plsc.subcore_barrier()
        pltpu.sync_copy(acc.at[pl.ds(s * RPT, RPT)],
                        out_hbm.at[c].at[pl.ds(s * RPT, RPT)])

    return edge_kernel


_deg_call = _make_edge_kernel(edge_split=True, gather=False, width=8)
_l1_call = _make_edge_kernel(edge_split=False, gather=True, width=16)
_l2_call = _make_edge_kernel(edge_split=True, gather=True, width=16)


def _tc1_body(x_ref, w1_ref, d0_ref, d1_ref, gab_ref, dis_ref):
    deg = d0_ref[...] + d1_ref[...] + 1.0
    dis = lax.rsqrt(jnp.maximum(deg, 1e-12))
    h = jnp.dot(x_ref[...], w1_ref[...], preferred_element_type=_f32)
    g = h * dis
    gab_ref[0] = g[:, :16]
    gab_ref[1] = g[:, 16:]
    dis_ref[...] = dis


def _tc2_body(s1_ref, gab_ref, dis_ref, b1a_ref, b1b_ref, w2a_ref, w2b_ref,
              g2_ref):
    dis = dis_ref[...]
    h1a = jnp.maximum(dis * (s1_ref[0] + gab_ref[0]) + b1a_ref[...], 0.0)
    h1b = jnp.maximum(dis * (s1_ref[1] + gab_ref[1]) + b1b_ref[...], 0.0)
    h2 = (jnp.dot(h1a, w2a_ref[...], preferred_element_type=_f32)
          + jnp.dot(h1b, w2b_ref[...], preferred_element_type=_f32))
    g2_ref[...] = h2 * dis


def _tc3_body(s2_ref, g2_ref, dis_ref, b2_ref, z_ref):
    z_ref[...] = (dis_ref[...] * (s2_ref[0] + s2_ref[1] + g2_ref[...])
                  + b2_ref[...])


_B = 3128
_GRID = NPAD // _B


def _rows3(i):
    return (0, i, 0)


def _rows2(i):
    return (i, 0)


def _full2(i):
    return (0, 0)


_blk3 = pl.BlockSpec((NC, _B, 16), _rows3)
_blk2 = pl.BlockSpec((_B, 16), _rows2)
_blk1 = pl.BlockSpec((_B, 1), _rows2)

_tc1 = pl.pallas_call(
    _tc1_body,
    grid=(_GRID,),
    in_specs=[pl.BlockSpec((_B, 6), _rows2),
              pl.BlockSpec((6, 32), _full2),
              _blk1, _blk1],
    out_specs=(_blk3, _blk1),
    out_shape=(jax.ShapeDtypeStruct((NC, NPAD, 16), _f32),
               jax.ShapeDtypeStruct((NPAD, 1), _f32)),
)
_tc2 = pl.pallas_call(
    _tc2_body,
    grid=(_GRID,),
    in_specs=[_blk3, _blk3, _blk1,
              pl.BlockSpec((1, 16), _full2), pl.BlockSpec((1, 16), _full2),
              pl.BlockSpec((16, 16), _full2), pl.BlockSpec((16, 16), _full2)],
    out_specs=_blk2,
    out_shape=jax.ShapeDtypeStruct((NPAD, 16), _f32),
)
_tc3 = pl.pallas_call(
    _tc3_body,
    grid=(_GRID,),
    in_specs=[_blk3, _blk2, _blk1, pl.BlockSpec((1, 16), _full2)],
    out_specs=_blk2,
    out_shape=jax.ShapeDtypeStruct((NPAD, 16), _f32),
)


@jax.jit
def kernel(x, edge_index, W1, b1, W2, b2):
    src = jnp.pad(edge_index[0], (0, EPAD - E), constant_values=N)
    dst = jnp.pad(edge_index[1], (0, EPAD - E), constant_values=N)
    src = src.reshape(ROWS, 128).astype(jnp.int32)
    dst = dst.reshape(ROWS, 128).astype(jnp.int32)
    xp = jnp.pad(x, ((0, NPAD - N), (0, 0)))
    zeros16 = jnp.zeros((NPAD, 16), _f32)
    zeros8 = jnp.zeros((NPAD, 8), _f32)
    ones = jnp.ones((1, 128, 8), _f32)

    degp = _deg_call(src, dst, ones, zeros8)
    gab, dis = _tc1(xp, W1, degp[0, :, :1], degp[1, :, :1])
    s1 = _l1_call(src, dst, gab, zeros16)
    g2 = _tc2(s1, gab, dis, b1[:16][None, :], b1[16:][None, :],
              W2[:16], W2[16:])
    s2 = _l2_call(src, dst, g2[None], zeros16)
    z = _tc3(s2, g2, dis, b2[None, :])
    return z[:N]
